# baseline (device time: 18932 ns/iter reference)
import jax
import jax.numpy as jnp
from jax import lax
from jax.experimental import pallas as pl
from jax.experimental.pallas import tpu as pltpu

N_DEV = 32
N_ROUNDS = 5


def kernel(x, dy, gamma):
    m, d = x.shape

    def body(x_ref, dy_ref, out_ref, acc_ref, recv_ref, send_sems, recv_sems):
        me = lax.axis_index("i")

        barrier_sem = pltpu.get_barrier_semaphore()
        for k in range(N_ROUNDS):
            p = me ^ (1 << k)
            pl.semaphore_signal(
                barrier_sem, inc=1,
                device_id=(p,), device_id_type=pl.DeviceIdType.MESH,
            )

        xv = x_ref[...].astype(jnp.float32)
        dyv = dy_ref[...].astype(jnp.float32)
        mu = jnp.mean(xv, axis=1, keepdims=True)
        var = jnp.mean((xv - mu) * (xv - mu), axis=1, keepdims=True)
        rstd = lax.rsqrt(var + 1e-5)
        xhat = (xv - mu) * rstd
        acc_ref[0, :] = jnp.sum(dyv * xhat, axis=0)
        acc_ref[1, :] = jnp.sum(dyv, axis=0)

        pl.semaphore_wait(barrier_sem, N_ROUNDS)

        for k in range(N_ROUNDS):
            p = me ^ (1 << k)
            rdma = pltpu.make_async_remote_copy(
                src_ref=acc_ref,
                dst_ref=recv_ref.at[k],
                send_sem=send_sems.at[k],
                recv_sem=recv_sems.at[k],
                device_id=p,
                device_id_type=pl.DeviceIdType.LOGICAL,
            )
            rdma.start()
            rdma.wait()
            acc_ref[...] = acc_ref[...] + recv_ref[k]

        out_ref[...] = acc_ref[...]

    return pl.pallas_call(
        body,
        out_shape=jax.ShapeDtypeStruct((2, d), jnp.float32),
        in_specs=[
            pl.BlockSpec(memory_space=pltpu.VMEM),
            pl.BlockSpec(memory_space=pltpu.VMEM),
        ],
        out_specs=pl.BlockSpec(memory_space=pltpu.VMEM),
        scratch_shapes=[
            pltpu.VMEM((2, d), jnp.float32),
            pltpu.VMEM((N_ROUNDS, 2, d), jnp.float32),
            pltpu.SemaphoreType.DMA((N_ROUNDS,)),
            pltpu.SemaphoreType.DMA((N_ROUNDS,)),
        ],
        compiler_params=pltpu.CompilerParams(collective_id=0),
    )(x, dy)


# device time: 14598 ns/iter; 1.2969x vs baseline; 1.2969x over previous
import jax
import jax.numpy as jnp
from jax import lax
from jax.experimental import pallas as pl
from jax.experimental.pallas import tpu as pltpu

N_DEV = 32
PLANE = 8
LINES = 4


def kernel(x, dy, gamma):
    m, d = x.shape

    def body(x_ref, dy_ref, out_ref, acc_ref, g1_ref, g2_ref,
             s1_send, s1_recv, s2_send, s2_recv):
        me = lax.axis_index("i")
        zme = me // PLANE
        j = me % PLANE

        barrier_sem = pltpu.get_barrier_semaphore()
        for off in range(1, PLANE):
            p = zme * PLANE + (j + off) % PLANE
            pl.semaphore_signal(barrier_sem, inc=1, device_id=(p,),
                                device_id_type=pl.DeviceIdType.MESH)
        for off in range(1, LINES):
            p = ((zme + off) % LINES) * PLANE + j
            pl.semaphore_signal(barrier_sem, inc=1, device_id=(p,),
                                device_id_type=pl.DeviceIdType.MESH)

        xv = x_ref[...].astype(jnp.float32)
        dyv = dy_ref[...].astype(jnp.float32)
        mu = jnp.mean(xv, axis=1, keepdims=True)
        var = jnp.mean((xv - mu) * (xv - mu), axis=1, keepdims=True)
        rstd = lax.rsqrt(var + 1e-5)
        xhat = (xv - mu) * rstd
        acc_ref[0, :] = jnp.sum(dyv * xhat, axis=0)
        acc_ref[1, :] = jnp.sum(dyv, axis=0)

        pl.semaphore_wait(barrier_sem, PLANE - 1 + LINES - 1)

        g1_ref[pl.ds(j, 1)] = acc_ref[...][None]
        sends1 = []
        for off in range(1, PLANE):
            jp = (j + off) % PLANE
            rdma = pltpu.make_async_remote_copy(
                src_ref=acc_ref,
                dst_ref=g1_ref.at[j],
                send_sem=s1_send.at[off],
                recv_sem=s1_recv.at[j],
                device_id=zme * PLANE + jp,
                device_id_type=pl.DeviceIdType.LOGICAL,
            )
            rdma.start()
            sends1.append(rdma)
        for off in range(1, PLANE):
            jp = (j + off) % PLANE
            recv = pltpu.make_async_remote_copy(
                src_ref=acc_ref, dst_ref=g1_ref.at[jp],
                send_sem=s1_send.at[off], recv_sem=s1_recv.at[jp],
                device_id=zme * PLANE + jp,
                device_id_type=pl.DeviceIdType.LOGICAL,
            )
            recv.wait_recv()
        plane_sum = jnp.sum(g1_ref[...], axis=0)

        g2_ref[pl.ds(zme, 1)] = plane_sum[None]
        sends2 = []
        for off in range(1, LINES):
            zp = (zme + off) % LINES
            rdma = pltpu.make_async_remote_copy(
                src_ref=g2_ref.at[zme],
                dst_ref=g2_ref.at[zme],
                send_sem=s2_send.at[off],
                recv_sem=s2_recv.at[zme],
                device_id=zp * PLANE + j,
                device_id_type=pl.DeviceIdType.LOGICAL,
            )
            rdma.start()
            sends2.append(rdma)
        for off in range(1, LINES):
            zp = (zme + off) % LINES
            recv = pltpu.make_async_remote_copy(
                src_ref=acc_ref, dst_ref=g2_ref.at[zp],
                send_sem=s2_send.at[off], recv_sem=s2_recv.at[zp],
                device_id=zp * PLANE + j,
                device_id_type=pl.DeviceIdType.LOGICAL,
            )
            recv.wait_recv()

        for rdma in sends1 + sends2:
            rdma.wait_send()

        out_ref[...] = jnp.sum(g2_ref[...], axis=0)

    return pl.pallas_call(
        body,
        out_shape=jax.ShapeDtypeStruct((2, d), jnp.float32),
        in_specs=[
            pl.BlockSpec(memory_space=pltpu.VMEM),
            pl.BlockSpec(memory_space=pltpu.VMEM),
        ],
        out_specs=pl.BlockSpec(memory_space=pltpu.VMEM),
        scratch_shapes=[
            pltpu.VMEM((2, d), jnp.float32),
            pltpu.VMEM((PLANE, 2, d), jnp.float32),
            pltpu.VMEM((LINES, 2, d), jnp.float32),
            pltpu.SemaphoreType.DMA((PLANE,)),
            pltpu.SemaphoreType.DMA((PLANE,)),
            pltpu.SemaphoreType.DMA((LINES,)),
            pltpu.SemaphoreType.DMA((LINES,)),
        ],
        compiler_params=pltpu.CompilerParams(collective_id=0),
    )(x, dy)
